# speedup vs baseline: 1.2193x; 1.2193x over previous
import functools

import jax
import jax.numpy as jnp
from jax import lax
from jax.experimental import pallas as pl
from jax.experimental.pallas import tpu as pltpu
from jax.experimental.pallas import tpu_sc as plsc

N_VOCAB = 100000
D_MODEL = 768
BOS = 0

NC = 2
NS = 16
NW = NC * NS

CHUNK = 64
N_OUT = 4 * 8193
N_MAIN = 32768
PER_W = N_MAIN // NW
CHUNKS_W = PER_W // CHUNK
N_PAD = 513 * CHUNK

_mesh = plsc.VectorSubcoreMesh(core_axis_name="c", subcore_axis_name="s")


@functools.partial(
    pl.kernel,
    out_type=jax.ShapeDtypeStruct((N_OUT, D_MODEL), jnp.float32),
    mesh=_mesh,
    scratch_types=[
        pltpu.VMEM((CHUNKS_W + 1, CHUNK), jnp.int32),
        pltpu.VMEM((CHUNK, D_MODEL), jnp.float32),
        pltpu.SemaphoreType.DMA,
    ],
)
def _gather_kernel(ids_hbm, table_hbm, out_hbm, idx_v, buf, sem):
    wid = lax.axis_index("s") * NC + lax.axis_index("c")
    row0 = wid * CHUNKS_W

    pltpu.sync_copy(ids_hbm.at[pl.ds(row0, CHUNKS_W)],
                    idx_v.at[pl.ds(0, CHUNKS_W)])

    for j in range(CHUNKS_W):
        pltpu.async_copy(table_hbm.at[idx_v.at[j]], buf, sem).wait()
        pltpu.sync_copy(buf, out_hbm.at[pl.ds(wid * PER_W + j * CHUNK, CHUNK)])

    @pl.when(wid == NW - 1)
    def _tail():
        pltpu.sync_copy(ids_hbm.at[pl.ds(N_MAIN // CHUNK, 1)],
                        idx_v.at[pl.ds(CHUNKS_W, 1)])
        pltpu.async_copy(table_hbm.at[idx_v.at[CHUNKS_W]], buf, sem).wait()
        pltpu.sync_copy(buf.at[pl.ds(0, N_OUT - N_MAIN)],
                        out_hbm.at[pl.ds(N_MAIN, N_OUT - N_MAIN)])


def kernel(input_ids, embedding):
    ids = jnp.pad(input_ids.astype(jnp.int32), ((0, 0), (1, 0)),
                  constant_values=BOS).reshape(-1)
    ids = jnp.pad(ids, (0, N_PAD - N_OUT), constant_values=0)
    out = _gather_kernel(ids.reshape(N_PAD // CHUNK, CHUNK), embedding)
    return out.reshape(input_ids.shape[0], input_ids.shape[1] + 1, D_MODEL)


# speedup vs baseline: 1.2357x; 1.0134x over previous
import functools

import jax
import jax.numpy as jnp
from jax import lax
from jax.experimental import pallas as pl
from jax.experimental.pallas import tpu as pltpu
from jax.experimental.pallas import tpu_sc as plsc

N_VOCAB = 100000
D_MODEL = 768
BOS = 0

NC = 2
NS = 16
NW = NC * NS

CHUNK = 64
N_OUT = 4 * 8193
N_MAIN = 32768
PER_W = N_MAIN // NW
CHUNKS_W = PER_W // CHUNK
N_PAD = 513 * CHUNK

_mesh = plsc.VectorSubcoreMesh(core_axis_name="c", subcore_axis_name="s")


@functools.partial(
    pl.kernel,
    out_type=jax.ShapeDtypeStruct((N_OUT, D_MODEL), jnp.float32),
    mesh=_mesh,
    scratch_types=[
        pltpu.VMEM((CHUNKS_W + 1, CHUNK), jnp.int32),
        pltpu.VMEM((CHUNK, D_MODEL), jnp.float32),
        pltpu.VMEM((CHUNK, D_MODEL), jnp.float32),
        pltpu.SemaphoreType.DMA,
        pltpu.SemaphoreType.DMA,
        pltpu.SemaphoreType.DMA,
        pltpu.SemaphoreType.DMA,
    ],
)
def _gather_kernel(ids_hbm, table_hbm, out_hbm, idx_v, buf0, buf1,
                   gsem0, gsem1, wsem0, wsem1):
    wid = lax.axis_index("s") * NC + lax.axis_index("c")
    row0 = wid * CHUNKS_W
    bufs = (buf0, buf1)
    gsems = (gsem0, gsem1)
    wsems = (wsem0, wsem1)

    pltpu.sync_copy(ids_hbm.at[pl.ds(row0, CHUNKS_W)],
                    idx_v.at[pl.ds(0, CHUNKS_W)])

    gathers = [None] * CHUNKS_W
    writes = [None] * CHUNKS_W
    gathers[0] = pltpu.async_copy(table_hbm.at[idx_v.at[0]], bufs[0], gsems[0])
    for j in range(CHUNKS_W):
        b = j % 2
        if j + 1 < CHUNKS_W:
            if j >= 1:
                writes[j - 1].wait()
            gathers[j + 1] = pltpu.async_copy(
                table_hbm.at[idx_v.at[j + 1]], bufs[1 - b], gsems[1 - b])
        gathers[j].wait()
        writes[j] = pltpu.async_copy(
            bufs[b], out_hbm.at[pl.ds(wid * PER_W + j * CHUNK, CHUNK)],
            wsems[b])
    writes[CHUNKS_W - 2].wait()
    writes[CHUNKS_W - 1].wait()

    @pl.when(wid == NW - 1)
    def _tail():
        pltpu.sync_copy(ids_hbm.at[pl.ds(N_MAIN // CHUNK, 1)],
                        idx_v.at[pl.ds(CHUNKS_W, 1)])
        pltpu.async_copy(table_hbm.at[idx_v.at[CHUNKS_W]], buf0, gsem0).wait()
        pltpu.sync_copy(buf0.at[pl.ds(0, N_OUT - N_MAIN)],
                        out_hbm.at[pl.ds(N_MAIN, N_OUT - N_MAIN)])


def kernel(input_ids, embedding):
    ids = jnp.pad(input_ids.astype(jnp.int32), ((0, 0), (1, 0)),
                  constant_values=BOS).reshape(-1)
    ids = jnp.pad(ids, (0, N_PAD - N_OUT), constant_values=0)
    out = _gather_kernel(ids.reshape(N_PAD // CHUNK, CHUNK), embedding)
    return out.reshape(input_ids.shape[0], input_ids.shape[1] + 1, D_MODEL)


# speedup vs baseline: 4.0183x; 3.2518x over previous
import functools

import jax
import jax.numpy as jnp
from jax import lax
from jax.experimental import pallas as pl
from jax.experimental.pallas import tpu as pltpu
from jax.experimental.pallas import tpu_sc as plsc

N_VOCAB = 100000
D_MODEL = 768
BOS = 0

NC = 2
NS = 16
NW = NC * NS

B = 4
T = 8193
CHUNK = 64
W_PER_B = NW // B
PER_W = 8192 // W_PER_B
CHUNKS_W = PER_W // CHUNK
T_PAD = 129 * CHUNK

_mesh = plsc.VectorSubcoreMesh(core_axis_name="c", subcore_axis_name="s")


@functools.partial(
    pl.kernel,
    out_type=jax.ShapeDtypeStruct((B, T, D_MODEL), jnp.float32),
    mesh=_mesh,
    scratch_types=[
        pltpu.VMEM((CHUNKS_W + 1, CHUNK), jnp.int32),
        pltpu.VMEM((CHUNK, D_MODEL), jnp.float32),
        pltpu.VMEM((CHUNK, D_MODEL), jnp.float32),
        pltpu.SemaphoreType.DMA,
        pltpu.SemaphoreType.DMA,
        pltpu.SemaphoreType.DMA,
        pltpu.SemaphoreType.DMA,
    ],
)
def _gather_kernel(ids_hbm, table_hbm, out_hbm, idx_v, buf0, buf1,
                   gsem0, gsem1, wsem0, wsem1):
    wid = lax.axis_index("s") * NC + lax.axis_index("c")
    b = wid // W_PER_B
    lane = wid % W_PER_B
    bufs = (buf0, buf1)
    gsems = (gsem0, gsem1)
    wsems = (wsem0, wsem1)

    pltpu.sync_copy(ids_hbm.at[b, pl.ds(lane * CHUNKS_W, CHUNKS_W)],
                    idx_v.at[pl.ds(0, CHUNKS_W)])

    gathers = [None] * CHUNKS_W
    writes = [None] * CHUNKS_W
    gathers[0] = pltpu.async_copy(table_hbm.at[idx_v.at[0]], bufs[0], gsems[0])
    for j in range(CHUNKS_W):
        k = j % 2
        if j + 1 < CHUNKS_W:
            if j >= 1:
                writes[j - 1].wait()
            gathers[j + 1] = pltpu.async_copy(
                table_hbm.at[idx_v.at[j + 1]], bufs[1 - k], gsems[1 - k])
        gathers[j].wait()
        writes[j] = pltpu.async_copy(
            bufs[k],
            out_hbm.at[b, pl.ds(lane * PER_W + j * CHUNK, CHUNK)],
            wsems[k])
    writes[CHUNKS_W - 2].wait()
    writes[CHUNKS_W - 1].wait()

    @pl.when(lane == W_PER_B - 1)
    def _tail():
        pltpu.sync_copy(ids_hbm.at[b, pl.ds(T_PAD // CHUNK - 1, 1)],
                        idx_v.at[pl.ds(CHUNKS_W, 1)])
        pltpu.async_copy(table_hbm.at[idx_v.at[CHUNKS_W]], buf0, gsem0).wait()
        pltpu.sync_copy(buf0.at[pl.ds(0, 1)],
                        out_hbm.at[b, pl.ds(W_PER_B * PER_W, 1)])


def kernel(input_ids, embedding):
    ids = jnp.pad(input_ids.astype(jnp.int32), ((0, 0), (1, 0)),
                  constant_values=BOS)
    ids = jnp.pad(ids, ((0, 0), (0, T_PAD - T)), constant_values=0)
    return _gather_kernel(ids.reshape(B, T_PAD // CHUNK, CHUNK), embedding)


# speedup vs baseline: 4.1251x; 1.0266x over previous
import functools

import jax
import jax.numpy as jnp
from jax import lax
from jax.experimental import pallas as pl
from jax.experimental.pallas import tpu as pltpu
from jax.experimental.pallas import tpu_sc as plsc

N_VOCAB = 100000
D_MODEL = 768
BOS = 0

NC = 2
NS = 16
NW = NC * NS

B = 4
T = 8193
CHUNK = 32
NBUF = 4
W_PER_B = NW // B
PER_W = 8192 // W_PER_B
CHUNKS_W = PER_W // CHUNK
T_PAD = T - 1 + CHUNK

_mesh = plsc.VectorSubcoreMesh(core_axis_name="c", subcore_axis_name="s")


@functools.partial(
    pl.kernel,
    out_type=jax.ShapeDtypeStruct((B, T, D_MODEL), jnp.float32),
    mesh=_mesh,
    scratch_types=[
        pltpu.VMEM((CHUNKS_W + 1, CHUNK), jnp.int32),
        [pltpu.VMEM((CHUNK, D_MODEL), jnp.float32) for _ in range(NBUF)],
        [pltpu.SemaphoreType.DMA for _ in range(NBUF)],
        [pltpu.SemaphoreType.DMA for _ in range(NBUF)],
    ],
)
def _gather_kernel(ids_hbm, table_hbm, out_hbm, idx_v, bufs, gsems, wsems):
    wid = lax.axis_index("s") * NC + lax.axis_index("c")
    b = wid // W_PER_B
    lane = wid % W_PER_B

    pltpu.sync_copy(ids_hbm.at[b, pl.ds(lane * CHUNKS_W, CHUNKS_W)],
                    idx_v.at[pl.ds(0, CHUNKS_W)])

    def gather(j):
        return pltpu.async_copy(table_hbm.at[idx_v.at[j]], bufs[j % NBUF],
                                gsems[j % NBUF])

    def write(j):
        return pltpu.async_copy(
            bufs[j % NBUF],
            out_hbm.at[b, pl.ds(lane * PER_W + j * CHUNK, CHUNK)],
            wsems[j % NBUF])

    gathers = [None] * CHUNKS_W
    writes = [None] * CHUNKS_W
    gathers[0] = gather(0)
    gathers[1] = gather(1)
    for j in range(CHUNKS_W):
        if j + 2 < CHUNKS_W:
            if j >= 2:
                writes[j - 2].wait()
            gathers[j + 2] = gather(j + 2)
        gathers[j].wait()
        writes[j] = write(j)
    for j in range(CHUNKS_W - 2, CHUNKS_W):
        writes[j].wait()

    @pl.when(lane == W_PER_B - 1)
    def _tail():
        pltpu.sync_copy(ids_hbm.at[b, pl.ds(T_PAD // CHUNK - 1, 1)],
                        idx_v.at[pl.ds(CHUNKS_W, 1)])
        pltpu.async_copy(table_hbm.at[idx_v.at[CHUNKS_W]], bufs[0],
                         gsems[0]).wait()
        pltpu.sync_copy(bufs[0].at[pl.ds(0, 1)],
                        out_hbm.at[b, pl.ds(W_PER_B * PER_W, 1)])


def kernel(input_ids, embedding):
    ids = jnp.pad(input_ids.astype(jnp.int32), ((0, 0), (1, 0)),
                  constant_values=BOS)
    ids = jnp.pad(ids, ((0, 0), (0, T_PAD - T)), constant_values=0)
    return _gather_kernel(ids.reshape(B, T_PAD // CHUNK, CHUNK), embedding)
